# Initial kernel scaffold; baseline (speedup 1.0000x reference)
import jax
import jax.numpy as jnp
from jax.experimental import pallas as pl


def kernel(x, edge_index, batch, eps0, W1_0, b1_0, W2_0, b2_0, g0, be0, eps1, W1_1, b1_1, W2_1, b2_1, g1, be1, eps2, W1_2, b1_2, W2_2, b2_2, g2, be2, fcW1, fcb1, fcW2, fcb2):
    raise NotImplementedError("write your pallas kernel here")


# speedup vs baseline: 5.7402x; 5.7402x over previous
import functools

import jax
import jax.numpy as jnp
import numpy as np
from jax import lax
from jax.experimental import pallas as pl
from jax.experimental.pallas import tpu as pltpu
from jax.experimental.pallas import tpu_sc as plsc

N = 10000
E = 320000
D = 128
G = 64
C = 16

NC = 2
NS = 16
NW = NC * NS
CHUNK = 128
NCHUNKS = E // CHUNK
STRIPE = 624
TAIL = N - NS * STRIPE

_BN_SCALE = float(1.0 / np.sqrt(1.0 + 1e-5))


def _sc_segsum_body(h_hbm, src_hbm, dst_hbm, zeros_hbm, out_hbm,
                    src_idx, dst_idx, rows, acc_sh, sem):
    c = lax.axis_index("c")
    s = lax.axis_index("s")
    wid = s * NC + c

    sbase = pl.multiple_of(s * STRIPE, 8)
    pltpu.sync_copy(zeros_hbm.at[pl.ds(sbase, STRIPE)],
                    acc_sh.at[pl.ds(sbase, STRIPE)])

    @pl.when(s == 0)
    def _():
        pltpu.sync_copy(zeros_hbm.at[pl.ds(NS * STRIPE, TAIL)],
                        acc_sh.at[pl.ds(NS * STRIPE, TAIL)])

    plsc.subcore_barrier()

    nchunks = (NCHUNKS - wid + NW - 1) // NW

    def body(j, carry):
        base = (wid + j * NW) * CHUNK
        pltpu.sync_copy(src_hbm.at[pl.ds(base, CHUNK)], src_idx)
        pltpu.sync_copy(dst_hbm.at[pl.ds(base, CHUNK)], dst_idx)
        pltpu.async_copy(h_hbm.at[src_idx], rows, sem).wait()
        pltpu.sync_copy(rows, acc_sh.at[dst_idx], add=True)
        return carry

    lax.fori_loop(0, nchunks, body, 0)
    plsc.subcore_barrier()

    pltpu.sync_copy(acc_sh.at[pl.ds(sbase, STRIPE)],
                    out_hbm.at[c, pl.ds(sbase, STRIPE)])

    @pl.when(s == 0)
    def _():
        pltpu.sync_copy(acc_sh.at[pl.ds(NS * STRIPE, TAIL)],
                        out_hbm.at[c, pl.ds(NS * STRIPE, TAIL)])


@functools.cache
def _sc_segsum_kernel():
    return pl.kernel(
        _sc_segsum_body,
        out_type=jax.ShapeDtypeStruct((NC, N, D), jnp.float32),
        mesh=plsc.VectorSubcoreMesh(core_axis_name="c", subcore_axis_name="s",
                                    num_cores=NC, num_subcores=NS),
        scratch_types=[
            pltpu.VMEM((CHUNK,), jnp.int32),
            pltpu.VMEM((CHUNK,), jnp.int32),
            pltpu.VMEM((CHUNK, D), jnp.float32),
            pltpu.VMEM_SHARED((N, D), jnp.float32),
            pltpu.SemaphoreType.DMA,
        ],
    )


def _sc_segsum(h, src, dst, zeros):
    return _sc_segsum_kernel()(h, src, dst, zeros)


def _tc_mlp_body(h_ref, p_ref, eps_ref, w1_ref, b1_ref, w2_ref, b2_ref,
                 g_ref, be_ref, out_ref):
    z = (1.0 + eps_ref[0, 0]) * h_ref[...] + p_ref[0] + p_ref[1]
    a = jnp.maximum(jnp.dot(z, w1_ref[...],
                            preferred_element_type=jnp.float32) + b1_ref[...], 0.0)
    a = jnp.maximum(jnp.dot(a, w2_ref[...],
                            preferred_element_type=jnp.float32) + b2_ref[...], 0.0)
    out_ref[...] = a * (_BN_SCALE * g_ref[...]) + be_ref[...]


_MLP_BLK = 2000


def _tc_mlp(h, parts, eps, w1, b1, w2, b2, g, be):
    grid = (N // _MLP_BLK,)
    full = lambda shape: pl.BlockSpec(shape, lambda i: (0,) * len(shape))
    return pl.pallas_call(
        _tc_mlp_body,
        grid=grid,
        in_specs=[
            pl.BlockSpec((_MLP_BLK, D), lambda i: (i, 0)),
            pl.BlockSpec((NC, _MLP_BLK, D), lambda i: (0, i, 0)),
            full((1, 1)), full((D, D)), full((1, D)), full((D, D)),
            full((1, D)), full((1, D)), full((1, D)),
        ],
        out_specs=pl.BlockSpec((_MLP_BLK, D), lambda i: (i, 0)),
        out_shape=jax.ShapeDtypeStruct((N, D), jnp.float32),
    )(h, parts, eps, w1, b1, w2, b2, g, be)


def _tc_pool_fc_body(h_ref, batch_ref, fw1_ref, fb1_ref, fw2_ref, fb2_ref,
                     out_ref):
    seg = lax.broadcasted_iota(jnp.int32, (G, N), 0)
    onehot_t = (seg == batch_ref[...]).astype(jnp.float32)
    sums = jnp.dot(onehot_t, h_ref[...],
                   preferred_element_type=jnp.float32)
    cnt = jnp.sum(onehot_t, axis=1, keepdims=True)
    pooled = sums / jnp.maximum(cnt, 1.0)
    a = jnp.maximum(jnp.dot(pooled, fw1_ref[...],
                            preferred_element_type=jnp.float32) + fb1_ref[...],
                    0.0)
    o = jnp.dot(a, fw2_ref[...],
                preferred_element_type=jnp.float32) + fb2_ref[...]
    m = jnp.max(o, axis=-1, keepdims=True)
    lse = jnp.log(jnp.sum(jnp.exp(o - m), axis=-1, keepdims=True)) + m
    out_ref[...] = o - lse


def _tc_pool_fc(h, batch2d, fw1, fb1, fw2, fb2):
    return pl.pallas_call(
        _tc_pool_fc_body,
        out_shape=jax.ShapeDtypeStruct((G, C), jnp.float32),
    )(h, batch2d, fw1, fb1, fw2, fb2)


def kernel(x, edge_index, batch, eps0, W1_0, b1_0, W2_0, b2_0, g0, be0,
           eps1, W1_1, b1_1, W2_1, b2_1, g1, be1,
           eps2, W1_2, b1_2, W2_2, b2_2, g2, be2,
           fcW1, fcb1, fcW2, fcb2):
    src = edge_index[0]
    dst = edge_index[1]
    zeros = jnp.zeros((N, D), jnp.float32)
    row = lambda v: v.reshape(1, D)

    h = x
    for eps, W1, b1, W2, b2, g, be in (
            (eps0, W1_0, b1_0, W2_0, b2_0, g0, be0),
            (eps1, W1_1, b1_1, W2_1, b2_1, g1, be1),
            (eps2, W1_2, b1_2, W2_2, b2_2, g2, be2)):
        parts = _sc_segsum(h, src, dst, zeros)
        h = _tc_mlp(h, parts, eps.reshape(1, 1), W1, row(b1), W2, row(b2),
                    row(g), row(be))

    return _tc_pool_fc(h, batch.reshape(1, N), fcW1, fcb1.reshape(1, D),
                       fcW2, fcb2.reshape(1, C))
